# baseline (device time: 94195 ns/iter reference)
import jax
import jax.numpy as jnp
from jax import lax
from jax.experimental import pallas as pl
from jax.experimental.pallas import tpu as pltpu

N_DEV = 4
SQ = 512
D = 1024
DH = 128
HQ_LOCAL = 8
SCALE = 0.08838834764831843


def kernel(x, Wq, Wo, Wk, Wv):
    def body(x_ref, wq_ref, wo_ref, wk_ref, wv_ref, out_ref,
             comm_ref, send_sems, recv_sems):
        my_i = lax.axis_index("i")
        right = lax.rem(my_i + 1, N_DEV)
        left = lax.rem(my_i + N_DEV - 1, N_DEV)

        barrier_sem = pltpu.get_barrier_semaphore()
        for nbr in (left, right):
            pl.semaphore_signal(
                barrier_sem, inc=1,
                device_id=(nbr,), device_id_type=pl.DeviceIdType.MESH,
            )
        pl.semaphore_wait(barrier_sem, 2)

        xm = x_ref[...]
        q_all = jnp.dot(xm, wq_ref[...], preferred_element_type=jnp.float32)

        kv_start = my_i * (2 * DH)
        k_all = jnp.dot(xm, wk_ref[:, pl.ds(kv_start, 2 * DH)],
                        preferred_element_type=jnp.float32)
        v_all = jnp.dot(xm, wv_ref[:, pl.ds(kv_start, 2 * DH)],
                        preferred_element_type=jnp.float32)

        outs = []
        for h in range(HQ_LOCAL):
            q = q_all[:, h * DH:(h + 1) * DH]
            g = h // 4
            k = k_all[:, g * DH:(g + 1) * DH]
            v = v_all[:, g * DH:(g + 1) * DH]
            s = lax.dot_general(
                q, k, (((1,), (1,)), ((), ())),
                preferred_element_type=jnp.float32,
            ) * SCALE
            m = jnp.max(s, axis=-1, keepdims=True)
            p = jnp.exp(s - m)
            l = jnp.sum(p, axis=-1, keepdims=True)
            o = jnp.dot(p, v, preferred_element_type=jnp.float32) / l
            outs.append(o)
        attn = jnp.concatenate(outs, axis=1)

        partial = jnp.dot(attn, wo_ref[...],
                          preferred_element_type=jnp.float32)

        comm_ref[0] = partial
        out_ref[...] = partial
        for h in range(N_DEV - 1):
            rdma = pltpu.make_async_remote_copy(
                src_ref=comm_ref.at[h],
                dst_ref=comm_ref.at[h + 1],
                send_sem=send_sems.at[h],
                recv_sem=recv_sems.at[h],
                device_id=(right,),
                device_id_type=pl.DeviceIdType.MESH,
            )
            rdma.start()
            rdma.wait()
            out_ref[...] += comm_ref[h + 1]

    out = pl.pallas_call(
        body,
        out_shape=jax.ShapeDtypeStruct((SQ, D), jnp.float32),
        in_specs=[pl.BlockSpec(memory_space=pltpu.VMEM)] * 5,
        out_specs=pl.BlockSpec(memory_space=pltpu.VMEM),
        scratch_shapes=[
            pltpu.VMEM((N_DEV, SQ, D), jnp.float32),
            pltpu.SemaphoreType.DMA((N_DEV - 1,)),
            pltpu.SemaphoreType.DMA((N_DEV - 1,)),
        ],
        compiler_params=pltpu.CompilerParams(collective_id=0),
    )(x[0], Wq, Wo, Wk, Wv)
    return out[None]


# device time: 47514 ns/iter; 1.9825x vs baseline; 1.9825x over previous
import jax
import jax.numpy as jnp
from jax import lax
from jax.experimental import pallas as pl
from jax.experimental.pallas import tpu as pltpu

N_DEV = 4
SQ = 512
D = 1024
DH = 128
HQ_LOCAL = 8
SCALE = 0.08838834764831843


CHUNK = SQ // N_DEV


def kernel(x, Wq, Wo, Wk, Wv):
    def body(x_ref, wq_ref, wo_ref, wk_ref, wv_ref, out_ref,
             stage_ref, rs_send_sems, rs_recv_sems,
             ag_send_sems, ag_recv_sems):
        my_i = lax.axis_index("i")
        peers = [lax.rem(my_i + d, N_DEV) for d in range(1, N_DEV)]

        barrier_sem = pltpu.get_barrier_semaphore()
        for p in peers:
            pl.semaphore_signal(
                barrier_sem, inc=1,
                device_id=(p,), device_id_type=pl.DeviceIdType.MESH,
            )
        pl.semaphore_wait(barrier_sem, N_DEV - 1)

        xm = x_ref[...]
        q_all = jnp.dot(xm, wq_ref[...], preferred_element_type=jnp.float32)

        kv_start = my_i * (2 * DH)
        k_all = jnp.dot(xm, wk_ref[:, pl.ds(kv_start, 2 * DH)],
                        preferred_element_type=jnp.float32)
        v_all = jnp.dot(xm, wv_ref[:, pl.ds(kv_start, 2 * DH)],
                        preferred_element_type=jnp.float32)

        outs = []
        for h in range(HQ_LOCAL):
            q = q_all[:, h * DH:(h + 1) * DH]
            g = h // 4
            k = k_all[:, g * DH:(g + 1) * DH]
            v = v_all[:, g * DH:(g + 1) * DH]
            s = lax.dot_general(
                q, k, (((1,), (1,)), ((), ())),
                preferred_element_type=jnp.float32,
            ) * SCALE
            m = jnp.max(s, axis=-1, keepdims=True)
            p = jnp.exp(s - m)
            l = jnp.sum(p, axis=-1, keepdims=True)
            o = jnp.dot(p, v, preferred_element_type=jnp.float32) / l
            outs.append(o)
        attn = jnp.concatenate(outs, axis=1)

        partial = jnp.dot(attn, wo_ref[...],
                          preferred_element_type=jnp.float32)

        out_ref[...] = partial

        rs = []
        for d in range(1, N_DEV):
            p = peers[d - 1]
            rdma = pltpu.make_async_remote_copy(
                src_ref=out_ref.at[pl.ds(p * CHUNK, CHUNK)],
                dst_ref=stage_ref.at[3 - d],
                send_sem=rs_send_sems.at[3 - d],
                recv_sem=rs_recv_sems.at[3 - d],
                device_id=(p,),
                device_id_type=pl.DeviceIdType.MESH,
            )
            rdma.start()
            rs.append(rdma)
        for rdma in rs:
            rdma.wait_recv()
        my_rows = pl.ds(my_i * CHUNK, CHUNK)
        out_ref[my_rows, :] = (
            out_ref[my_rows, :]
            + stage_ref[0] + stage_ref[1] + stage_ref[2]
        )

        ag = []
        for d in range(1, N_DEV):
            p = peers[d - 1]
            rdma = pltpu.make_async_remote_copy(
                src_ref=out_ref.at[my_rows],
                dst_ref=out_ref.at[my_rows],
                send_sem=ag_send_sems.at[3 - d],
                recv_sem=ag_recv_sems.at[3 - d],
                device_id=(p,),
                device_id_type=pl.DeviceIdType.MESH,
            )
            rdma.start()
            ag.append(rdma)
        for rdma in ag:
            rdma.wait_recv()
        for rdma in rs + ag:
            rdma.wait_send()

    out = pl.pallas_call(
        body,
        out_shape=jax.ShapeDtypeStruct((SQ, D), jnp.float32),
        in_specs=[pl.BlockSpec(memory_space=pltpu.VMEM)] * 5,
        out_specs=pl.BlockSpec(memory_space=pltpu.VMEM),
        scratch_shapes=[
            pltpu.VMEM((N_DEV - 1, CHUNK, D), jnp.float32),
            pltpu.SemaphoreType.DMA((N_DEV - 1,)),
            pltpu.SemaphoreType.DMA((N_DEV - 1,)),
            pltpu.SemaphoreType.DMA((N_DEV - 1,)),
            pltpu.SemaphoreType.DMA((N_DEV - 1,)),
        ],
        compiler_params=pltpu.CompilerParams(collective_id=0),
    )(x[0], Wq, Wo, Wk, Wv)
    return out[None]


# device time: 46970 ns/iter; 2.0054x vs baseline; 1.0116x over previous
import jax
import jax.numpy as jnp
from jax import lax
from jax.experimental import pallas as pl
from jax.experimental.pallas import tpu as pltpu

N_DEV = 4
SQ = 512
D = 1024
DH = 128
HQ_LOCAL = 8
SCALE = 0.08838834764831843


CHUNK = SQ // N_DEV


def kernel(x, Wq, Wo, Wk, Wv):
    def body(x_ref, wq_ref, wo_ref, wk_ref, wv_ref, out_ref,
             stage_ref, rs_send_sems, rs_recv_sems,
             ag_send_sems, ag_recv_sems):
        my_i = lax.axis_index("i")
        peers = [lax.rem(my_i + d, N_DEV) for d in range(1, N_DEV)]

        barrier_sem = pltpu.get_barrier_semaphore()
        for p in peers:
            pl.semaphore_signal(
                barrier_sem, inc=1,
                device_id=(p,), device_id_type=pl.DeviceIdType.MESH,
            )
        pl.semaphore_wait(barrier_sem, N_DEV - 1)

        xm = x_ref[...]
        kv_start = my_i * (2 * DH)
        k_all = jnp.dot(xm, wk_ref[:, pl.ds(kv_start, 2 * DH)],
                        preferred_element_type=jnp.float32)
        v_all = jnp.dot(xm, wv_ref[:, pl.ds(kv_start, 2 * DH)],
                        preferred_element_type=jnp.float32)

        def compute_chunk(c):
            xc = x_ref[pl.ds(c * CHUNK, CHUNK), :]
            qc = jnp.dot(xc, wq_ref[...],
                         preferred_element_type=jnp.float32)
            outs = []
            for h in range(HQ_LOCAL):
                q = qc[:, h * DH:(h + 1) * DH]
                g = h // 4
                k = k_all[:, g * DH:(g + 1) * DH]
                v = v_all[:, g * DH:(g + 1) * DH]
                s = lax.dot_general(
                    q, k, (((1,), (1,)), ((), ())),
                    preferred_element_type=jnp.float32,
                ) * SCALE
                m = jnp.max(s, axis=-1, keepdims=True)
                p = jnp.exp(s - m)
                l = jnp.sum(p, axis=-1, keepdims=True)
                o = jnp.dot(p, v, preferred_element_type=jnp.float32) / l
                outs.append(o)
            attn = jnp.concatenate(outs, axis=1)
            return jnp.dot(attn, wo_ref[...],
                           preferred_element_type=jnp.float32)

        rs = []
        for d in range(1, N_DEV):
            p = peers[d - 1]
            p_rows = pl.ds(p * CHUNK, CHUNK)
            out_ref[p_rows, :] = compute_chunk(p)
            rdma = pltpu.make_async_remote_copy(
                src_ref=out_ref.at[p_rows],
                dst_ref=stage_ref.at[3 - d],
                send_sem=rs_send_sems.at[3 - d],
                recv_sem=rs_recv_sems.at[3 - d],
                device_id=(p,),
                device_id_type=pl.DeviceIdType.MESH,
            )
            rdma.start()
            rs.append(rdma)
        own = compute_chunk(my_i)
        for rdma in rs:
            rdma.wait_recv()
        my_rows = pl.ds(my_i * CHUNK, CHUNK)
        out_ref[my_rows, :] = (
            own + stage_ref[0] + stage_ref[1] + stage_ref[2]
        )

        ag = []
        for d in range(1, N_DEV):
            p = peers[d - 1]
            rdma = pltpu.make_async_remote_copy(
                src_ref=out_ref.at[my_rows],
                dst_ref=out_ref.at[my_rows],
                send_sem=ag_send_sems.at[3 - d],
                recv_sem=ag_recv_sems.at[3 - d],
                device_id=(p,),
                device_id_type=pl.DeviceIdType.MESH,
            )
            rdma.start()
            ag.append(rdma)
        for rdma in ag:
            rdma.wait_recv()
        for rdma in rs + ag:
            rdma.wait_send()

    out = pl.pallas_call(
        body,
        out_shape=jax.ShapeDtypeStruct((SQ, D), jnp.float32),
        in_specs=[pl.BlockSpec(memory_space=pltpu.VMEM)] * 5,
        out_specs=pl.BlockSpec(memory_space=pltpu.VMEM),
        scratch_shapes=[
            pltpu.VMEM((N_DEV - 1, CHUNK, D), jnp.float32),
            pltpu.SemaphoreType.DMA((N_DEV - 1,)),
            pltpu.SemaphoreType.DMA((N_DEV - 1,)),
            pltpu.SemaphoreType.DMA((N_DEV - 1,)),
            pltpu.SemaphoreType.DMA((N_DEV - 1,)),
        ],
        compiler_params=pltpu.CompilerParams(collective_id=0),
    )(x[0], Wq, Wo, Wk, Wv)
    return out[None]


# device time: 37136 ns/iter; 2.5365x vs baseline; 1.2648x over previous
import jax
import jax.numpy as jnp
from jax import lax
from jax.experimental import pallas as pl
from jax.experimental.pallas import tpu as pltpu

N_DEV = 4
SQ = 512
D = 1024
DH = 128
HQ_LOCAL = 8
SCALE = 0.08838834764831843


CHUNK = SQ // N_DEV


def kernel(x, Wq, Wo, Wk, Wv):
    def body(x_ref, wq_ref, wo_ref, wk_ref, wv_ref, out_ref,
             rs_src_ref, stage_ref, ag_src_ref, ag_stage_ref,
             rs_send_sems, rs_recv_sems, ag_send_sems, ag_recv_sems):
        my_i = lax.axis_index("i")
        peers = [lax.rem(my_i + d, N_DEV) for d in range(1, N_DEV)]

        barrier_sem = pltpu.get_barrier_semaphore()
        for p in peers:
            pl.semaphore_signal(
                barrier_sem, inc=1,
                device_id=(p,), device_id_type=pl.DeviceIdType.MESH,
            )
        pl.semaphore_wait(barrier_sem, N_DEV - 1)

        bf16 = jnp.bfloat16
        xm = x_ref[...].astype(bf16)
        wq_b = wq_ref[...].astype(bf16)
        wo_b = wo_ref[...].astype(bf16)
        kv_start = my_i * (2 * DH)
        k_all = jnp.dot(xm, wk_ref[:, pl.ds(kv_start, 2 * DH)].astype(bf16),
                        preferred_element_type=jnp.float32).astype(bf16)
        v_all = jnp.dot(xm, wv_ref[:, pl.ds(kv_start, 2 * DH)].astype(bf16),
                        preferred_element_type=jnp.float32).astype(bf16)

        def compute_chunk(c):
            xc = x_ref[pl.ds(c * CHUNK, CHUNK), :].astype(bf16)
            qc = jnp.dot(xc, wq_b,
                         preferred_element_type=jnp.float32).astype(bf16)
            outs = []
            for h in range(HQ_LOCAL):
                q = qc[:, h * DH:(h + 1) * DH]
                g = h // 4
                k = k_all[:, g * DH:(g + 1) * DH]
                v = v_all[:, g * DH:(g + 1) * DH]
                s = lax.dot_general(
                    q, k, (((1,), (1,)), ((), ())),
                    preferred_element_type=jnp.float32,
                ) * SCALE
                m = jnp.max(s, axis=-1, keepdims=True)
                p = jnp.exp(s - m)
                l = jnp.sum(p, axis=-1, keepdims=True)
                o = jnp.dot(p.astype(bf16), v,
                            preferred_element_type=jnp.float32) / l
                outs.append(o.astype(bf16))
            attn = jnp.concatenate(outs, axis=1)
            return jnp.dot(attn, wo_b,
                           preferred_element_type=jnp.float32)

        rs = []
        for d in range(1, N_DEV):
            p = peers[d - 1]
            rs_src_ref[3 - d] = compute_chunk(p).astype(bf16)
            rdma = pltpu.make_async_remote_copy(
                src_ref=rs_src_ref.at[3 - d],
                dst_ref=stage_ref.at[3 - d],
                send_sem=rs_send_sems.at[3 - d],
                recv_sem=rs_recv_sems.at[3 - d],
                device_id=(p,),
                device_id_type=pl.DeviceIdType.MESH,
            )
            rdma.start()
            rs.append(rdma)
        own = compute_chunk(my_i)
        for rdma in rs:
            rdma.wait_recv()
        my_rows = pl.ds(my_i * CHUNK, CHUNK)
        reduced = (
            own
            + stage_ref[0].astype(jnp.float32)
            + stage_ref[1].astype(jnp.float32)
            + stage_ref[2].astype(jnp.float32)
        )
        out_ref[my_rows, :] = reduced
        ag_src_ref[...] = reduced.astype(bf16)

        ag = []
        for d in range(1, N_DEV):
            p = peers[d - 1]
            rdma = pltpu.make_async_remote_copy(
                src_ref=ag_src_ref,
                dst_ref=ag_stage_ref.at[3 - d],
                send_sem=ag_send_sems.at[3 - d],
                recv_sem=ag_recv_sems.at[3 - d],
                device_id=(p,),
                device_id_type=pl.DeviceIdType.MESH,
            )
            rdma.start()
            ag.append(rdma)
        for rdma in ag:
            rdma.wait_recv()
        for s in range(N_DEV - 1):
            out_ref[pl.ds(peers[s] * CHUNK, CHUNK), :] = (
                ag_stage_ref[s].astype(jnp.float32)
            )
        for rdma in rs + ag:
            rdma.wait_send()

    out = pl.pallas_call(
        body,
        out_shape=jax.ShapeDtypeStruct((SQ, D), jnp.float32),
        in_specs=[pl.BlockSpec(memory_space=pltpu.VMEM)] * 5,
        out_specs=pl.BlockSpec(memory_space=pltpu.VMEM),
        scratch_shapes=[
            pltpu.VMEM((N_DEV - 1, CHUNK, D), jnp.bfloat16),
            pltpu.VMEM((N_DEV - 1, CHUNK, D), jnp.bfloat16),
            pltpu.VMEM((CHUNK, D), jnp.bfloat16),
            pltpu.VMEM((N_DEV - 1, CHUNK, D), jnp.bfloat16),
            pltpu.SemaphoreType.DMA((N_DEV - 1,)),
            pltpu.SemaphoreType.DMA((N_DEV - 1,)),
            pltpu.SemaphoreType.DMA((N_DEV - 1,)),
            pltpu.SemaphoreType.DMA((N_DEV - 1,)),
        ],
        compiler_params=pltpu.CompilerParams(collective_id=0),
    )(x[0], Wq, Wo, Wk, Wv)
    return out[None]
